# Initial kernel scaffold; baseline (speedup 1.0000x reference)
import jax
import jax.numpy as jnp
from jax.experimental import pallas as pl


def kernel(brain_wave, W_enc, b_enc, codebook, W_dec, b_dec):
    raise NotImplementedError("write your pallas kernel here")


# speedup vs baseline: 1.3509x; 1.3509x over previous
import jax
import jax.numpy as jnp
from jax.experimental import pallas as pl
from jax.experimental.pallas import tpu as pltpu

_B, _T, _C = 8, 4096, 64
_P = 4
_D = 256
_K = 1024
_N = (_B * _T) // _P
_BLK = 512
_GRID = _N // _BLK


def _vq_body(x_ref, we_ref, be_ref, cb_ref, wd_ref, bd_ref,
             out_ref, dec_cb_ref, e2_ref):
    i = pl.program_id(0)

    @pl.when(i == 0)
    def _():
        cb = cb_ref[...]
        dec_cb_ref[...] = (
            jnp.dot(cb, wd_ref[...], preferred_element_type=jnp.float32)
            + bd_ref[...]
        )
        e2_ref[...] = jnp.sum(cb * cb, axis=1, keepdims=True).T

    x = x_ref[...]
    z = jnp.dot(x, we_ref[...], preferred_element_type=jnp.float32) + be_ref[...]
    s = jax.lax.dot_general(
        z, cb_ref[...], (((1,), (1,)), ((), ())),
        preferred_element_type=jnp.float32)
    z2 = jnp.sum(z * z, axis=1, keepdims=True)
    dists = z2 - 2.0 * s + e2_ref[...]
    minval = jnp.min(dists, axis=1, keepdims=True)
    iota = jax.lax.broadcasted_iota(jnp.int32, (_BLK, _K), 1)
    idx = jnp.min(jnp.where(dists == minval, iota, _K), axis=1)
    one_hot = (iota == idx[:, None]).astype(jnp.float32)
    out_ref[...] = jnp.dot(one_hot, dec_cb_ref[...],
                           preferred_element_type=jnp.float32)


def kernel(brain_wave, W_enc, b_enc, codebook, W_dec, b_dec):
    x = brain_wave.reshape(_N, _P * _C)
    out = pl.pallas_call(
        _vq_body,
        grid=(_GRID,),
        in_specs=[
            pl.BlockSpec((_BLK, _P * _C), lambda i: (i, 0)),
            pl.BlockSpec((_P * _C, _D), lambda i: (0, 0)),
            pl.BlockSpec((1, _D), lambda i: (0, 0)),
            pl.BlockSpec((_K, _D), lambda i: (0, 0)),
            pl.BlockSpec((_D, _P * _C), lambda i: (0, 0)),
            pl.BlockSpec((1, _P * _C), lambda i: (0, 0)),
        ],
        out_specs=pl.BlockSpec((_BLK, _P * _C), lambda i: (i, 0)),
        out_shape=jax.ShapeDtypeStruct((_N, _P * _C), jnp.float32),
        scratch_shapes=[
            pltpu.VMEM((_K, _P * _C), jnp.float32),
            pltpu.VMEM((1, _K), jnp.float32),
        ],
    )(x, W_enc, b_enc.reshape(1, _D), codebook, W_dec,
      b_dec.reshape(1, _P * _C))
    return out.reshape(_B, _T, _C)
